# baseline (speedup 1.0000x reference)
import functools

import jax
import jax.numpy as jnp
from jax import lax
from jax.experimental import pallas as pl
from jax.experimental.pallas import tpu as pltpu
from jax.experimental.pallas import tpu_sc as plsc

N = 10000
E = 320000
D = 128
N_PAD = 10240
NC, NS, L = 2, 16, 16
NW = NC * NS
K = 64
E_TOT = E + N
N_CHUNKS = -(-E_TOT // (NW * K))
N_CHUNKS += N_CHUNKS % 2
T_PER_TILE = N_CHUNKS * K
E_PAD = T_PER_TILE * NW
E_HBM = E_PAD + K
ROWS_PER_TILE = N_PAD // NS


_BLK = 1024
_GRID = N_PAD // _BLK


def _tc_prep_body(x_ref, w_ref, asr_ref, adr_ref, h_ref, aa_ref):
    h = jnp.dot(x_ref[...], w_ref[...], preferred_element_type=jnp.float32)
    h_ref[...] = h
    av = jnp.sum(h * asr_ref[...][None, :], axis=1)
    dv = jnp.sum(h * adr_ref[...][None, :], axis=1)
    aa_ref[...] = jnp.concatenate(
        [av[None], dv[None], jnp.zeros((6, av.shape[0]), jnp.float32)], axis=0)


def _tc_prep(x, W, a_s, a_d):
    return pl.pallas_call(
        _tc_prep_body,
        grid=(_GRID,),
        in_specs=[
            pl.BlockSpec((_BLK, D), lambda i: (i, 0)),
            pl.BlockSpec((D, D), lambda i: (0, 0)),
            pl.BlockSpec((D,), lambda i: (0,)),
            pl.BlockSpec((D,), lambda i: (0,)),
        ],
        out_specs=[
            pl.BlockSpec((_BLK, D), lambda i: (i, 0)),
            pl.BlockSpec((8, _BLK), lambda i: (0, i)),
        ],
        out_shape=[
            jax.ShapeDtypeStruct((N_PAD, D), jnp.float32),
            jax.ShapeDtypeStruct((8, N_PAD), jnp.float32),
        ],
    )(x, W, a_s, a_d)


def _combine(acc_ref, den_ref, b_ref):
    accsum = acc_ref[0] + acc_ref[1]
    den = jnp.sum(den_ref[...], axis=0)
    return accsum / (den[:, None] + 1e-16) + b_ref[...][None, :]


def _tc_mid_body(acc_ref, den_ref, b_ref, w_ref, asr_ref, adr_ref,
                 h_ref, aa_ref):
    out1 = _combine(acc_ref, den_ref, b_ref)
    nrm = jnp.sqrt(jnp.sum(out1 * out1, axis=1, keepdims=True))
    out1 = out1 / jnp.maximum(nrm, 1e-12)
    out1 = jnp.maximum(out1, 0.0)
    h2 = jnp.dot(out1, w_ref[...], preferred_element_type=jnp.float32)
    h_ref[...] = h2
    av = jnp.sum(h2 * asr_ref[...][None, :], axis=1)
    dv = jnp.sum(h2 * adr_ref[...][None, :], axis=1)
    aa_ref[...] = jnp.concatenate(
        [av[None], dv[None], jnp.zeros((6, av.shape[0]), jnp.float32)], axis=0)


def _tc_mid(acc, den, b, W, a_s, a_d):
    return pl.pallas_call(
        _tc_mid_body,
        grid=(_GRID,),
        in_specs=[
            pl.BlockSpec((NC, _BLK, D), lambda i: (0, i, 0)),
            pl.BlockSpec((NC, _BLK), lambda i: (0, i)),
            pl.BlockSpec((D,), lambda i: (0,)),
            pl.BlockSpec((D, D), lambda i: (0, 0)),
            pl.BlockSpec((D,), lambda i: (0,)),
            pl.BlockSpec((D,), lambda i: (0,)),
        ],
        out_specs=[
            pl.BlockSpec((_BLK, D), lambda i: (i, 0)),
            pl.BlockSpec((8, _BLK), lambda i: (0, i)),
        ],
        out_shape=[
            jax.ShapeDtypeStruct((N_PAD, D), jnp.float32),
            jax.ShapeDtypeStruct((8, N_PAD), jnp.float32),
        ],
    )(acc, den, b, W, a_s, a_d)


def _tc_final_body(acc_ref, den_ref, b_ref, o_ref):
    o_ref[...] = _combine(acc_ref, den_ref, b_ref)


def _tc_final(acc, den, b):
    return pl.pallas_call(
        _tc_final_body,
        grid=(_GRID,),
        in_specs=[
            pl.BlockSpec((NC, _BLK, D), lambda i: (0, i, 0)),
            pl.BlockSpec((NC, _BLK), lambda i: (0, i)),
            pl.BlockSpec((D,), lambda i: (0,)),
        ],
        out_specs=pl.BlockSpec((_BLK, D), lambda i: (i, 0)),
        out_shape=jax.ShapeDtypeStruct((N_PAD, D), jnp.float32),
    )(acc, den, b)


def _edge_pass_kernel(h_hbm, aa_hbm, src_hbm, dst_hbm,
                      acc_hbm, den_hbm,
                      acc_sh, den_sh, as_tab, ad_tab,
                      si_0, si_1, di_0, di_1, sdi_0, sdi_1,
                      rows_0, rows_1, w_0, w_1,
                      sem_g0, sem_g1, sem_s0, sem_s1, sem_d0, sem_d1,
                      sem_a0, sem_a1, sem_w0, sem_w1):
    cid = lax.axis_index("c")
    sid = lax.axis_index("s")
    wid = sid * NC + cid
    base = wid * T_PER_TILE

    zero16 = jnp.zeros((L,), jnp.float32)
    rows = (rows_0, rows_1)
    sis = (si_0, si_1)
    dis = (di_0, di_1)
    sdis = (sdi_0, sdi_1)
    w_v = (w_0, w_1)
    sems_g = (sem_g0, sem_g1)
    sems_s = (sem_s0, sem_s1)
    sems_d = (sem_d0, sem_d1)
    sems_a = (sem_a0, sem_a1)
    sems_w = (sem_w0, sem_w1)

    def _zrow(j, _):
        for c in range(D // L):
            rows_0[j, pl.ds(c * L, L)] = zero16
        return 0
    lax.fori_loop(0, K, _zrow, 0)
    for g in range(K // L):
        w_0[pl.ds(g * L, L)] = zero16

    for kk in range(ROWS_PER_TILE // K):
        off = sid * ROWS_PER_TILE + kk * K
        pltpu.sync_copy(rows_0, acc_sh.at[pl.ds(off, K)])
        pltpu.sync_copy(w_0, den_sh.at[pl.ds(off, K)])

    pltpu.sync_copy(aa_hbm.at[0], as_tab)
    pltpu.sync_copy(aa_hbm.at[1], ad_tab)

    plsc.subcore_barrier()

    def _issue_idx(c, b):
        sl = pl.ds(base + c * K, K)
        pltpu.async_copy(src_hbm.at[sl], sis[b], sems_s[b])
        pltpu.async_copy(dst_hbm.at[sl], dis[b], sems_d[b])

    def _wait_idx(c, b):
        sl = pl.ds(base + c * K, K)
        pltpu.make_async_copy(src_hbm.at[sl], sis[b], sems_s[b]).wait()
        pltpu.make_async_copy(dst_hbm.at[sl], dis[b], sems_d[b]).wait()

    def _issue_gather(c, b):
        pltpu.async_copy(h_hbm.at[sis[b]], rows[b], sems_g[b])

    def _wait_gather(c, b):
        pltpu.make_async_copy(h_hbm.at[sis[b]], rows[b], sems_g[b]).wait()

    def _process(c, b):
        for g in range(K // L):
            s16 = sis[b][pl.ds(g * L, L)]
            d16 = dis[b][pl.ds(g * L, L)]
            e = plsc.load_gather(as_tab, [s16]) + plsc.load_gather(ad_tab, [d16])
            e = jnp.maximum(e, 0.2 * e)
            w = jnp.exp(e)
            w_v[b][pl.ds(g * L, L)] = w
            sdis[b][pl.ds(g * L, L)] = d16

        pltpu.async_copy(w_v[b], den_sh.at[sdis[b]], sems_w[b], add=True)

        rv = rows[b]
        wv = w_v[b]

        @plsc.parallel_loop(0, K, 1, unroll=8)
        def _scale(jj):
            j16 = jnp.full((L,), 0, jnp.int32) + jj
            wj = plsc.load_gather(wv, [j16])
            for c_ in range(D // L):
                rv[jj, pl.ds(c_ * L, L)] = rv[jj, pl.ds(c_ * L, L)] * wj

        pltpu.async_copy(rv, acc_sh.at[sdis[b]], sems_a[b], add=True)

    def _wait_scatters(c, b):
        pltpu.make_async_copy(w_v[b], den_sh.at[sdis[b]], sems_w[b]).wait()
        pltpu.make_async_copy(rows[b], acc_sh.at[sdis[b]], sems_a[b]).wait()

    _issue_idx(0, 0)
    _issue_idx(1, 1)
    _wait_idx(0, 0)
    _issue_gather(0, 0)
    _wait_idx(1, 1)
    _issue_gather(1, 1)
    _wait_gather(0, 0)
    _process(0, 0)
    _issue_idx(2, 0)

    def _step(c, b):
        nb = 1 - b
        _wait_scatters(c - 1, nb)
        _wait_idx(c + 1, nb)
        _issue_gather(c + 1, nb)
        _wait_gather(c, b)
        _process(c, b)
        _issue_idx(c + 2, b)

    def _pair(k, _):
        _step(2 * k + 1, 1)
        _step(2 * k + 2, 0)
        return 0

    lax.fori_loop(0, (N_CHUNKS - 2) // 2, _pair, 0)

    c_last = N_CHUNKS - 1
    _wait_scatters(c_last - 1, 0)
    _wait_gather(c_last, 1)
    _process(c_last, 1)
    _wait_scatters(c_last, 1)
    _wait_idx(N_CHUNKS, 0)

    plsc.subcore_barrier()

    stripe = pl.ds(sid * ROWS_PER_TILE, ROWS_PER_TILE)
    pltpu.sync_copy(acc_sh.at[stripe], acc_hbm.at[cid, stripe])
    pltpu.sync_copy(den_sh.at[stripe], den_hbm.at[cid, stripe])


def _edge_pass(h, aa, src, dst):
    mesh = plsc.VectorSubcoreMesh(core_axis_name="c", subcore_axis_name="s",
                                  num_cores=NC, num_subcores=NS)
    kern = pl.kernel(
        _edge_pass_kernel,
        out_type=(
            jax.ShapeDtypeStruct((NC, N_PAD, D), jnp.float32),
            jax.ShapeDtypeStruct((NC, N_PAD), jnp.float32),
        ),
        mesh=mesh,
        compiler_params=pltpu.CompilerParams(needs_layout_passes=False),
        scratch_types=(
            pltpu.VMEM_SHARED((N_PAD, D), jnp.float32),
            pltpu.VMEM_SHARED((N_PAD,), jnp.float32),
            pltpu.VMEM((N_PAD,), jnp.float32),
            pltpu.VMEM((N_PAD,), jnp.float32),
            pltpu.VMEM((K,), jnp.int32),
            pltpu.VMEM((K,), jnp.int32),
            pltpu.VMEM((K,), jnp.int32),
            pltpu.VMEM((K,), jnp.int32),
            pltpu.VMEM((K,), jnp.int32),
            pltpu.VMEM((K,), jnp.int32),
            pltpu.VMEM((K, D), jnp.float32),
            pltpu.VMEM((K, D), jnp.float32),
            pltpu.VMEM((K,), jnp.float32),
            pltpu.VMEM((K,), jnp.float32),
            pltpu.SemaphoreType.DMA,
            pltpu.SemaphoreType.DMA,
            pltpu.SemaphoreType.DMA,
            pltpu.SemaphoreType.DMA,
            pltpu.SemaphoreType.DMA,
            pltpu.SemaphoreType.DMA,
            pltpu.SemaphoreType.DMA,
            pltpu.SemaphoreType.DMA,
            pltpu.SemaphoreType.DMA,
            pltpu.SemaphoreType.DMA,
        ),
    )
    return kern(h, aa, src, dst)


def kernel(x_note, edge_index, edge_attr, W1, a_src1, a_dst1, b1,
           W2, a_src2, a_dst2, b2):
    del edge_attr
    x_pad = jnp.zeros((N_PAD, D), jnp.float32).at[:N].set(x_note)

    loops = jnp.arange(N, dtype=jnp.int32)
    pad = jnp.full((E_HBM - E_TOT,), N_PAD - 1, jnp.int32)
    src = jnp.concatenate([edge_index[0].astype(jnp.int32), loops, pad])
    dst = jnp.concatenate([edge_index[1].astype(jnp.int32), loops, pad])

    h1, aa1 = _tc_prep(x_pad, W1, a_src1, a_dst1)
    acc1, den1 = _edge_pass(h1, aa1, src, dst)
    h2, aa2 = _tc_mid(acc1, den1, b1, W2, a_src2, a_dst2)
    acc2, den2 = _edge_pass(h2, aa2, src, dst)
    out = _tc_final(acc2, den2, b2)
    return out[:N]


# speedup vs baseline: 1.0020x; 1.0020x over previous
import functools

import jax
import jax.numpy as jnp
from jax import lax
from jax.experimental import pallas as pl
from jax.experimental.pallas import tpu as pltpu
from jax.experimental.pallas import tpu_sc as plsc

N = 10000
E = 320000
D = 128
N_PAD = 10240
NC, NS, L = 2, 16, 16
NW = NC * NS
K = 64
E_TOT = E + N
N_CHUNKS = -(-E_TOT // (NW * K))
N_CHUNKS += N_CHUNKS % 2
T_PER_TILE = N_CHUNKS * K
E_PAD = T_PER_TILE * NW
E_HBM = E_PAD + K
ROWS_PER_TILE = N_PAD // NS


_BLK = 1024
_GRID = N_PAD // _BLK


def _tc_prep_body(x_ref, w_ref, asr_ref, adr_ref, h_ref, aa_ref):
    h = jnp.dot(x_ref[...], w_ref[...], preferred_element_type=jnp.float32)
    h_ref[...] = h
    av = jnp.sum(h * asr_ref[...][None, :], axis=1)
    dv = jnp.sum(h * adr_ref[...][None, :], axis=1)
    aa_ref[...] = jnp.concatenate(
        [av[None], dv[None], jnp.zeros((6, av.shape[0]), jnp.float32)], axis=0)


def _tc_prep(x, W, a_s, a_d):
    return pl.pallas_call(
        _tc_prep_body,
        grid=(_GRID,),
        in_specs=[
            pl.BlockSpec((_BLK, D), lambda i: (i, 0)),
            pl.BlockSpec((D, D), lambda i: (0, 0)),
            pl.BlockSpec((D,), lambda i: (0,)),
            pl.BlockSpec((D,), lambda i: (0,)),
        ],
        out_specs=[
            pl.BlockSpec((_BLK, D), lambda i: (i, 0)),
            pl.BlockSpec((8, _BLK), lambda i: (0, i)),
        ],
        out_shape=[
            jax.ShapeDtypeStruct((N_PAD, D), jnp.float32),
            jax.ShapeDtypeStruct((8, N_PAD), jnp.float32),
        ],
    )(x, W, a_s, a_d)


def _combine(acc_ref, den_ref, b_ref):
    accsum = acc_ref[0] + acc_ref[1]
    den = jnp.sum(den_ref[...], axis=0)
    return accsum / (den[:, None] + 1e-16) + b_ref[...][None, :]


def _tc_mid_body(acc_ref, den_ref, b_ref, w_ref, asr_ref, adr_ref,
                 h_ref, aa_ref):
    out1 = _combine(acc_ref, den_ref, b_ref)
    nrm = jnp.sqrt(jnp.sum(out1 * out1, axis=1, keepdims=True))
    out1 = out1 / jnp.maximum(nrm, 1e-12)
    out1 = jnp.maximum(out1, 0.0)
    h2 = jnp.dot(out1, w_ref[...], preferred_element_type=jnp.float32)
    h_ref[...] = h2
    av = jnp.sum(h2 * asr_ref[...][None, :], axis=1)
    dv = jnp.sum(h2 * adr_ref[...][None, :], axis=1)
    aa_ref[...] = jnp.concatenate(
        [av[None], dv[None], jnp.zeros((6, av.shape[0]), jnp.float32)], axis=0)


def _tc_mid(acc, den, b, W, a_s, a_d):
    return pl.pallas_call(
        _tc_mid_body,
        grid=(_GRID,),
        in_specs=[
            pl.BlockSpec((NC, _BLK, D), lambda i: (0, i, 0)),
            pl.BlockSpec((NC, _BLK), lambda i: (0, i)),
            pl.BlockSpec((D,), lambda i: (0,)),
            pl.BlockSpec((D, D), lambda i: (0, 0)),
            pl.BlockSpec((D,), lambda i: (0,)),
            pl.BlockSpec((D,), lambda i: (0,)),
        ],
        out_specs=[
            pl.BlockSpec((_BLK, D), lambda i: (i, 0)),
            pl.BlockSpec((8, _BLK), lambda i: (0, i)),
        ],
        out_shape=[
            jax.ShapeDtypeStruct((N_PAD, D), jnp.float32),
            jax.ShapeDtypeStruct((8, N_PAD), jnp.float32),
        ],
    )(acc, den, b, W, a_s, a_d)


def _tc_final_body(acc_ref, den_ref, b_ref, o_ref):
    o_ref[...] = _combine(acc_ref, den_ref, b_ref)


def _tc_final(acc, den, b):
    return pl.pallas_call(
        _tc_final_body,
        grid=(_GRID,),
        in_specs=[
            pl.BlockSpec((NC, _BLK, D), lambda i: (0, i, 0)),
            pl.BlockSpec((NC, _BLK), lambda i: (0, i)),
            pl.BlockSpec((D,), lambda i: (0,)),
        ],
        out_specs=pl.BlockSpec((_BLK, D), lambda i: (i, 0)),
        out_shape=jax.ShapeDtypeStruct((N_PAD, D), jnp.float32),
    )(acc, den, b)


def _edge_pass_kernel(h_hbm, aa_hbm, src_hbm, dst_hbm,
                      acc_hbm, den_hbm,
                      acc_sh, den_sh, as_tab, ad_tab,
                      si_0, si_1, di_0, di_1, sdi_0, sdi_1,
                      rows_0, rows_1, w_0, w_1,
                      sem_g0, sem_g1, sem_s0, sem_s1, sem_d0, sem_d1,
                      sem_a0, sem_a1, sem_w0, sem_w1):
    cid = lax.axis_index("c")
    sid = lax.axis_index("s")
    wid = sid * NC + cid
    base = wid * T_PER_TILE

    zero16 = jnp.zeros((L,), jnp.float32)
    rows = (rows_0, rows_1)
    sis = (si_0, si_1)
    dis = (di_0, di_1)
    sdis = (sdi_0, sdi_1)
    w_v = (w_0, w_1)
    sems_g = (sem_g0, sem_g1)
    sems_s = (sem_s0, sem_s1)
    sems_d = (sem_d0, sem_d1)
    sems_a = (sem_a0, sem_a1)
    sems_w = (sem_w0, sem_w1)

    def _zrow(j, _):
        for c in range(D // L):
            rows_0[j, pl.ds(c * L, L)] = zero16
        return 0
    lax.fori_loop(0, K, _zrow, 0)
    for g in range(K // L):
        w_0[pl.ds(g * L, L)] = zero16

    for kk in range(ROWS_PER_TILE // K):
        off = sid * ROWS_PER_TILE + kk * K
        pltpu.sync_copy(rows_0, acc_sh.at[pl.ds(off, K)])
        pltpu.sync_copy(w_0, den_sh.at[pl.ds(off, K)])

    pltpu.sync_copy(aa_hbm.at[0], as_tab)
    pltpu.sync_copy(aa_hbm.at[1], ad_tab)

    plsc.subcore_barrier()

    def _issue_idx(c, b):
        sl = pl.ds(base + c * K, K)
        pltpu.async_copy(src_hbm.at[sl], sis[b], sems_s[b])
        pltpu.async_copy(dst_hbm.at[sl], dis[b], sems_d[b])

    def _wait_idx(c, b):
        sl = pl.ds(base + c * K, K)
        pltpu.make_async_copy(src_hbm.at[sl], sis[b], sems_s[b]).wait()
        pltpu.make_async_copy(dst_hbm.at[sl], dis[b], sems_d[b]).wait()

    def _issue_gather(c, b):
        pltpu.async_copy(h_hbm.at[sis[b]], rows[b], sems_g[b])

    def _wait_gather(c, b):
        pltpu.make_async_copy(h_hbm.at[sis[b]], rows[b], sems_g[b]).wait()

    def _process(c, b):
        for g in range(K // L):
            s16 = sis[b][pl.ds(g * L, L)]
            d16 = dis[b][pl.ds(g * L, L)]
            e = plsc.load_gather(as_tab, [s16]) + plsc.load_gather(ad_tab, [d16])
            e = jnp.maximum(e, 0.2 * e)
            w = jnp.exp(e)
            w_v[b][pl.ds(g * L, L)] = w
            sdis[b][pl.ds(g * L, L)] = d16

        pltpu.async_copy(w_v[b], den_sh.at[sdis[b]], sems_w[b], add=True)

        rv = rows[b]
        wv = w_v[b]

        @plsc.parallel_loop(0, K, 1, unroll=4)
        def _scale(jj):
            j16 = jnp.full((L,), 0, jnp.int32) + jj
            wj = plsc.load_gather(wv, [j16])
            for c_ in range(D // L):
                rv[jj, pl.ds(c_ * L, L)] = rv[jj, pl.ds(c_ * L, L)] * wj

        pltpu.async_copy(rv, acc_sh.at[sdis[b]], sems_a[b], add=True)

    def _wait_scatters(c, b):
        pltpu.make_async_copy(w_v[b], den_sh.at[sdis[b]], sems_w[b]).wait()
        pltpu.make_async_copy(rows[b], acc_sh.at[sdis[b]], sems_a[b]).wait()

    _issue_idx(0, 0)
    _issue_idx(1, 1)
    _wait_idx(0, 0)
    _issue_gather(0, 0)
    _wait_idx(1, 1)
    _issue_gather(1, 1)
    _wait_gather(0, 0)
    _process(0, 0)
    _issue_idx(2, 0)

    def _step(c, b):
        nb = 1 - b
        _wait_scatters(c - 1, nb)
        _wait_idx(c + 1, nb)
        _issue_gather(c + 1, nb)
        _wait_gather(c, b)
        _process(c, b)
        _issue_idx(c + 2, b)

    def _pair(k, _):
        _step(2 * k + 1, 1)
        _step(2 * k + 2, 0)
        return 0

    lax.fori_loop(0, (N_CHUNKS - 2) // 2, _pair, 0)

    c_last = N_CHUNKS - 1
    _wait_scatters(c_last - 1, 0)
    _wait_gather(c_last, 1)
    _process(c_last, 1)
    _wait_scatters(c_last, 1)
    _wait_idx(N_CHUNKS, 0)

    plsc.subcore_barrier()

    stripe = pl.ds(sid * ROWS_PER_TILE, ROWS_PER_TILE)
    pltpu.sync_copy(acc_sh.at[stripe], acc_hbm.at[cid, stripe])
    pltpu.sync_copy(den_sh.at[stripe], den_hbm.at[cid, stripe])


def _edge_pass(h, aa, src, dst):
    mesh = plsc.VectorSubcoreMesh(core_axis_name="c", subcore_axis_name="s",
                                  num_cores=NC, num_subcores=NS)
    kern = pl.kernel(
        _edge_pass_kernel,
        out_type=(
            jax.ShapeDtypeStruct((NC, N_PAD, D), jnp.float32),
            jax.ShapeDtypeStruct((NC, N_PAD), jnp.float32),
        ),
        mesh=mesh,
        compiler_params=pltpu.CompilerParams(needs_layout_passes=False),
        scratch_types=(
            pltpu.VMEM_SHARED((N_PAD, D), jnp.float32),
            pltpu.VMEM_SHARED((N_PAD,), jnp.float32),
            pltpu.VMEM((N_PAD,), jnp.float32),
            pltpu.VMEM((N_PAD,), jnp.float32),
            pltpu.VMEM((K,), jnp.int32),
            pltpu.VMEM((K,), jnp.int32),
            pltpu.VMEM((K,), jnp.int32),
            pltpu.VMEM((K,), jnp.int32),
            pltpu.VMEM((K,), jnp.int32),
            pltpu.VMEM((K,), jnp.int32),
            pltpu.VMEM((K, D), jnp.float32),
            pltpu.VMEM((K, D), jnp.float32),
            pltpu.VMEM((K,), jnp.float32),
            pltpu.VMEM((K,), jnp.float32),
            pltpu.SemaphoreType.DMA,
            pltpu.SemaphoreType.DMA,
            pltpu.SemaphoreType.DMA,
            pltpu.SemaphoreType.DMA,
            pltpu.SemaphoreType.DMA,
            pltpu.SemaphoreType.DMA,
            pltpu.SemaphoreType.DMA,
            pltpu.SemaphoreType.DMA,
            pltpu.SemaphoreType.DMA,
            pltpu.SemaphoreType.DMA,
        ),
    )
    return kern(h, aa, src, dst)


def kernel(x_note, edge_index, edge_attr, W1, a_src1, a_dst1, b1,
           W2, a_src2, a_dst2, b2):
    del edge_attr
    x_pad = jnp.zeros((N_PAD, D), jnp.float32).at[:N].set(x_note)

    loops = jnp.arange(N, dtype=jnp.int32)
    pad = jnp.full((E_HBM - E_TOT,), N_PAD - 1, jnp.int32)
    src = jnp.concatenate([edge_index[0].astype(jnp.int32), loops, pad])
    dst = jnp.concatenate([edge_index[1].astype(jnp.int32), loops, pad])

    h1, aa1 = _tc_prep(x_pad, W1, a_src1, a_dst1)
    acc1, den1 = _edge_pass(h1, aa1, src, dst)
    h2, aa2 = _tc_mid(acc1, den1, b1, W2, a_src2, a_dst2)
    acc2, den2 = _edge_pass(h2, aa2, src, dst)
    out = _tc_final(acc2, den2, b2)
    return out[:N]


# speedup vs baseline: 1.0275x; 1.0254x over previous
import functools

import jax
import jax.numpy as jnp
from jax import lax
from jax.experimental import pallas as pl
from jax.experimental.pallas import tpu as pltpu
from jax.experimental.pallas import tpu_sc as plsc

N = 10000
E = 320000
D = 128
N_PAD = 10240
NC, NS, L = 2, 16, 16
NW = NC * NS
K = 64
E_TOT = E + N
N_CHUNKS = -(-E_TOT // (NW * K))
N_CHUNKS += N_CHUNKS % 2
T_PER_TILE = N_CHUNKS * K
E_PAD = T_PER_TILE * NW
E_HBM = E_PAD + K
ROWS_PER_TILE = N_PAD // NS


_BLK = 1024
_GRID = N_PAD // _BLK


def _tc_prep_body(x_ref, w_ref, asr_ref, adr_ref, h_ref, aa_ref):
    h = jnp.dot(x_ref[...], w_ref[...], preferred_element_type=jnp.float32)
    h_ref[...] = h
    av = jnp.sum(h * asr_ref[...][None, :], axis=1)
    dv = jnp.sum(h * adr_ref[...][None, :], axis=1)
    aa_ref[...] = jnp.concatenate(
        [av[None], dv[None], jnp.zeros((6, av.shape[0]), jnp.float32)], axis=0)


def _tc_prep(x, W, a_s, a_d):
    return pl.pallas_call(
        _tc_prep_body,
        grid=(_GRID,),
        in_specs=[
            pl.BlockSpec((_BLK, D), lambda i: (i, 0)),
            pl.BlockSpec((D, D), lambda i: (0, 0)),
            pl.BlockSpec((D,), lambda i: (0,)),
            pl.BlockSpec((D,), lambda i: (0,)),
        ],
        out_specs=[
            pl.BlockSpec((_BLK, D), lambda i: (i, 0)),
            pl.BlockSpec((8, _BLK), lambda i: (0, i)),
        ],
        out_shape=[
            jax.ShapeDtypeStruct((N_PAD, D), jnp.float32),
            jax.ShapeDtypeStruct((8, N_PAD), jnp.float32),
        ],
    )(x, W, a_s, a_d)


def _combine(acc_ref, den_ref, b_ref):
    accsum = acc_ref[0] + acc_ref[1]
    den = jnp.sum(den_ref[...], axis=0)
    return accsum / (den[:, None] + 1e-16) + b_ref[...][None, :]


def _tc_mid_body(acc_ref, den_ref, b_ref, w_ref, asr_ref, adr_ref,
                 h_ref, aa_ref):
    out1 = _combine(acc_ref, den_ref, b_ref)
    nrm = jnp.sqrt(jnp.sum(out1 * out1, axis=1, keepdims=True))
    out1 = out1 / jnp.maximum(nrm, 1e-12)
    out1 = jnp.maximum(out1, 0.0)
    h2 = jnp.dot(out1, w_ref[...], preferred_element_type=jnp.float32)
    h_ref[...] = h2
    av = jnp.sum(h2 * asr_ref[...][None, :], axis=1)
    dv = jnp.sum(h2 * adr_ref[...][None, :], axis=1)
    aa_ref[...] = jnp.concatenate(
        [av[None], dv[None], jnp.zeros((6, av.shape[0]), jnp.float32)], axis=0)


def _tc_mid(acc, den, b, W, a_s, a_d):
    return pl.pallas_call(
        _tc_mid_body,
        grid=(_GRID,),
        in_specs=[
            pl.BlockSpec((NC, _BLK, D), lambda i: (0, i, 0)),
            pl.BlockSpec((NC, _BLK), lambda i: (0, i)),
            pl.BlockSpec((D,), lambda i: (0,)),
            pl.BlockSpec((D, D), lambda i: (0, 0)),
            pl.BlockSpec((D,), lambda i: (0,)),
            pl.BlockSpec((D,), lambda i: (0,)),
        ],
        out_specs=[
            pl.BlockSpec((_BLK, D), lambda i: (i, 0)),
            pl.BlockSpec((8, _BLK), lambda i: (0, i)),
        ],
        out_shape=[
            jax.ShapeDtypeStruct((N_PAD, D), jnp.float32),
            jax.ShapeDtypeStruct((8, N_PAD), jnp.float32),
        ],
    )(acc, den, b, W, a_s, a_d)


def _tc_final_body(acc_ref, den_ref, b_ref, o_ref):
    o_ref[...] = _combine(acc_ref, den_ref, b_ref)


def _tc_final(acc, den, b):
    return pl.pallas_call(
        _tc_final_body,
        grid=(_GRID,),
        in_specs=[
            pl.BlockSpec((NC, _BLK, D), lambda i: (0, i, 0)),
            pl.BlockSpec((NC, _BLK), lambda i: (0, i)),
            pl.BlockSpec((D,), lambda i: (0,)),
        ],
        out_specs=pl.BlockSpec((_BLK, D), lambda i: (i, 0)),
        out_shape=jax.ShapeDtypeStruct((N_PAD, D), jnp.float32),
    )(acc, den, b)


def _edge_pass_kernel(h_hbm, aa_hbm, src_hbm, dst_hbm,
                      acc_hbm, den_hbm,
                      acc_sh, den_sh, as_tab, ad_tab,
                      si_0, si_1, di_0, di_1, sdi_0, sdi_1,
                      rows_0, rows_1, w_0, w_1,
                      sem_g0, sem_g1, sem_s0, sem_s1, sem_d0, sem_d1,
                      sem_a0, sem_a1, sem_w0, sem_w1):
    cid = lax.axis_index("c")
    sid = lax.axis_index("s")
    wid = sid * NC + cid
    base = wid * T_PER_TILE

    zero16 = jnp.zeros((L,), jnp.float32)
    rows = (rows_0, rows_1)
    sis = (si_0, si_1)
    dis = (di_0, di_1)
    sdis = (sdi_0, sdi_1)
    w_v = (w_0, w_1)
    sems_g = (sem_g0, sem_g1)
    sems_s = (sem_s0, sem_s1)
    sems_d = (sem_d0, sem_d1)
    sems_a = (sem_a0, sem_a1)
    sems_w = (sem_w0, sem_w1)

    def _zrow(j, _):
        for c in range(D // L):
            rows_0[j, pl.ds(c * L, L)] = zero16
        return 0
    lax.fori_loop(0, K, _zrow, 0)
    for g in range(K // L):
        w_0[pl.ds(g * L, L)] = zero16

    for kk in range(ROWS_PER_TILE // K):
        off = sid * ROWS_PER_TILE + kk * K
        pltpu.async_copy(rows_0, acc_sh.at[pl.ds(off, K)], sem_a0)
        pltpu.async_copy(w_0, den_sh.at[pl.ds(off, K)], sem_w0)
    pltpu.async_copy(aa_hbm.at[0], as_tab, sem_g0)
    pltpu.async_copy(aa_hbm.at[1], ad_tab, sem_g1)
    for kk in range(ROWS_PER_TILE // K):
        off = sid * ROWS_PER_TILE + kk * K
        pltpu.make_async_copy(rows_0, acc_sh.at[pl.ds(off, K)], sem_a0).wait()
        pltpu.make_async_copy(w_0, den_sh.at[pl.ds(off, K)], sem_w0).wait()
    pltpu.make_async_copy(aa_hbm.at[0], as_tab, sem_g0).wait()
    pltpu.make_async_copy(aa_hbm.at[1], ad_tab, sem_g1).wait()

    plsc.subcore_barrier()

    def _issue_idx(c, b):
        sl = pl.ds(base + c * K, K)
        pltpu.async_copy(src_hbm.at[sl], sis[b], sems_s[b])
        pltpu.async_copy(dst_hbm.at[sl], dis[b], sems_d[b])

    def _wait_idx(c, b):
        sl = pl.ds(base + c * K, K)
        pltpu.make_async_copy(src_hbm.at[sl], sis[b], sems_s[b]).wait()
        pltpu.make_async_copy(dst_hbm.at[sl], dis[b], sems_d[b]).wait()

    def _issue_gather(c, b):
        pltpu.async_copy(h_hbm.at[sis[b]], rows[b], sems_g[b])

    def _wait_gather(c, b):
        pltpu.make_async_copy(h_hbm.at[sis[b]], rows[b], sems_g[b]).wait()

    def _process(c, b):
        wvb, sdb, sib, dib = w_v[b], sdis[b], sis[b], dis[b]

        @plsc.parallel_loop(0, K // L, 1, unroll=4)
        def _weights(g):
            s16 = sib[pl.ds(g * L, L)]
            d16 = dib[pl.ds(g * L, L)]
            e = plsc.load_gather(as_tab, [s16]) + plsc.load_gather(ad_tab, [d16])
            e = jnp.maximum(e, 0.2 * e)
            w = jnp.exp(e)
            wvb[pl.ds(g * L, L)] = w
            sdb[pl.ds(g * L, L)] = d16

        pltpu.async_copy(w_v[b], den_sh.at[sdis[b]], sems_w[b], add=True)

        rv = rows[b]
        wv = w_v[b]

        @plsc.parallel_loop(0, K, 1, unroll=4)
        def _scale(jj):
            j16 = jnp.full((L,), 0, jnp.int32) + jj
            wj = plsc.load_gather(wv, [j16])
            for c_ in range(D // L):
                rv[jj, pl.ds(c_ * L, L)] = rv[jj, pl.ds(c_ * L, L)] * wj

        pltpu.async_copy(rv, acc_sh.at[sdis[b]], sems_a[b], add=True)

    def _wait_scatters(c, b):
        pltpu.make_async_copy(w_v[b], den_sh.at[sdis[b]], sems_w[b]).wait()
        pltpu.make_async_copy(rows[b], acc_sh.at[sdis[b]], sems_a[b]).wait()

    _issue_idx(0, 0)
    _issue_idx(1, 1)
    _wait_idx(0, 0)
    _issue_gather(0, 0)
    _wait_idx(1, 1)
    _issue_gather(1, 1)
    _wait_gather(0, 0)
    _process(0, 0)
    _issue_idx(2, 0)

    def _step(c, b):
        nb = 1 - b
        _wait_scatters(c - 1, nb)
        _wait_idx(c + 1, nb)
        _issue_gather(c + 1, nb)
        _wait_gather(c, b)
        _process(c, b)
        _issue_idx(c + 2, b)

    def _pair(k, _):
        _step(2 * k + 1, 1)
        _step(2 * k + 2, 0)
        return 0

    lax.fori_loop(0, (N_CHUNKS - 2) // 2, _pair, 0)

    c_last = N_CHUNKS - 1
    _wait_scatters(c_last - 1, 0)
    _wait_gather(c_last, 1)
    _process(c_last, 1)
    _wait_scatters(c_last, 1)
    _wait_idx(N_CHUNKS, 0)

    plsc.subcore_barrier()

    stripe = pl.ds(sid * ROWS_PER_TILE, ROWS_PER_TILE)
    pltpu.sync_copy(acc_sh.at[stripe], acc_hbm.at[cid, stripe])
    pltpu.sync_copy(den_sh.at[stripe], den_hbm.at[cid, stripe])


def _edge_pass(h, aa, src, dst):
    mesh = plsc.VectorSubcoreMesh(core_axis_name="c", subcore_axis_name="s",
                                  num_cores=NC, num_subcores=NS)
    kern = pl.kernel(
        _edge_pass_kernel,
        out_type=(
            jax.ShapeDtypeStruct((NC, N_PAD, D), jnp.float32),
            jax.ShapeDtypeStruct((NC, N_PAD), jnp.float32),
        ),
        mesh=mesh,
        compiler_params=pltpu.CompilerParams(needs_layout_passes=False),
        scratch_types=(
            pltpu.VMEM_SHARED((N_PAD, D), jnp.float32),
            pltpu.VMEM_SHARED((N_PAD,), jnp.float32),
            pltpu.VMEM((N_PAD,), jnp.float32),
            pltpu.VMEM((N_PAD,), jnp.float32),
            pltpu.VMEM((K,), jnp.int32),
            pltpu.VMEM((K,), jnp.int32),
            pltpu.VMEM((K,), jnp.int32),
            pltpu.VMEM((K,), jnp.int32),
            pltpu.VMEM((K,), jnp.int32),
            pltpu.VMEM((K,), jnp.int32),
            pltpu.VMEM((K, D), jnp.float32),
            pltpu.VMEM((K, D), jnp.float32),
            pltpu.VMEM((K,), jnp.float32),
            pltpu.VMEM((K,), jnp.float32),
            pltpu.SemaphoreType.DMA,
            pltpu.SemaphoreType.DMA,
            pltpu.SemaphoreType.DMA,
            pltpu.SemaphoreType.DMA,
            pltpu.SemaphoreType.DMA,
            pltpu.SemaphoreType.DMA,
            pltpu.SemaphoreType.DMA,
            pltpu.SemaphoreType.DMA,
            pltpu.SemaphoreType.DMA,
            pltpu.SemaphoreType.DMA,
        ),
    )
    return kern(h, aa, src, dst)


def kernel(x_note, edge_index, edge_attr, W1, a_src1, a_dst1, b1,
           W2, a_src2, a_dst2, b2):
    del edge_attr
    x_pad = jnp.zeros((N_PAD, D), jnp.float32).at[:N].set(x_note)

    loops = jnp.arange(N, dtype=jnp.int32)
    pad = jnp.full((E_HBM - E_TOT,), N_PAD - 1, jnp.int32)
    src = jnp.concatenate([edge_index[0].astype(jnp.int32), loops, pad])
    dst = jnp.concatenate([edge_index[1].astype(jnp.int32), loops, pad])

    h1, aa1 = _tc_prep(x_pad, W1, a_src1, a_dst1)
    acc1, den1 = _edge_pass(h1, aa1, src, dst)
    h2, aa2 = _tc_mid(acc1, den1, b1, W2, a_src2, a_dst2)
    acc2, den2 = _edge_pass(h2, aa2, src, dst)
    out = _tc_final(acc2, den2, b2)
    return out[:N]


# speedup vs baseline: 1.0793x; 1.0504x over previous
import functools

import jax
import jax.numpy as jnp
from jax import lax
from jax.experimental import pallas as pl
from jax.experimental.pallas import tpu as pltpu
from jax.experimental.pallas import tpu_sc as plsc

N = 10000
E = 320000
D = 128
N_PAD = 10240
NC, NS, L = 2, 16, 16
NW = NC * NS
K = 96
E_TOT = E + N
N_CHUNKS = -(-E_TOT // (NW * K))
N_CHUNKS += N_CHUNKS % 2
T_PER_TILE = N_CHUNKS * K
E_PAD = T_PER_TILE * NW
E_HBM = E_PAD + K
ROWS_PER_TILE = N_PAD // NS


_BLK = 1024
_GRID = N_PAD // _BLK


def _tc_prep_body(x_ref, w_ref, asr_ref, adr_ref, h_ref, aa_ref):
    h = jnp.dot(x_ref[...], w_ref[...], preferred_element_type=jnp.float32)
    h_ref[...] = h
    av = jnp.sum(h * asr_ref[...][None, :], axis=1)
    dv = jnp.sum(h * adr_ref[...][None, :], axis=1)
    aa_ref[...] = jnp.concatenate(
        [av[None], dv[None], jnp.zeros((6, av.shape[0]), jnp.float32)], axis=0)


def _tc_prep(x, W, a_s, a_d):
    return pl.pallas_call(
        _tc_prep_body,
        grid=(_GRID,),
        in_specs=[
            pl.BlockSpec((_BLK, D), lambda i: (i, 0)),
            pl.BlockSpec((D, D), lambda i: (0, 0)),
            pl.BlockSpec((D,), lambda i: (0,)),
            pl.BlockSpec((D,), lambda i: (0,)),
        ],
        out_specs=[
            pl.BlockSpec((_BLK, D), lambda i: (i, 0)),
            pl.BlockSpec((8, _BLK), lambda i: (0, i)),
        ],
        out_shape=[
            jax.ShapeDtypeStruct((N_PAD, D), jnp.float32),
            jax.ShapeDtypeStruct((8, N_PAD), jnp.float32),
        ],
    )(x, W, a_s, a_d)


def _combine(acc_ref, den_ref, b_ref):
    accsum = acc_ref[0] + acc_ref[1]
    den = jnp.sum(den_ref[...], axis=0)
    return accsum / (den[:, None] + 1e-16) + b_ref[...][None, :]


def _tc_mid_body(acc_ref, den_ref, b_ref, w_ref, asr_ref, adr_ref,
                 h_ref, aa_ref):
    out1 = _combine(acc_ref, den_ref, b_ref)
    nrm = jnp.sqrt(jnp.sum(out1 * out1, axis=1, keepdims=True))
    out1 = out1 / jnp.maximum(nrm, 1e-12)
    out1 = jnp.maximum(out1, 0.0)
    h2 = jnp.dot(out1, w_ref[...], preferred_element_type=jnp.float32)
    h_ref[...] = h2
    av = jnp.sum(h2 * asr_ref[...][None, :], axis=1)
    dv = jnp.sum(h2 * adr_ref[...][None, :], axis=1)
    aa_ref[...] = jnp.concatenate(
        [av[None], dv[None], jnp.zeros((6, av.shape[0]), jnp.float32)], axis=0)


def _tc_mid(acc, den, b, W, a_s, a_d):
    return pl.pallas_call(
        _tc_mid_body,
        grid=(_GRID,),
        in_specs=[
            pl.BlockSpec((NC, _BLK, D), lambda i: (0, i, 0)),
            pl.BlockSpec((NC, _BLK), lambda i: (0, i)),
            pl.BlockSpec((D,), lambda i: (0,)),
            pl.BlockSpec((D, D), lambda i: (0, 0)),
            pl.BlockSpec((D,), lambda i: (0,)),
            pl.BlockSpec((D,), lambda i: (0,)),
        ],
        out_specs=[
            pl.BlockSpec((_BLK, D), lambda i: (i, 0)),
            pl.BlockSpec((8, _BLK), lambda i: (0, i)),
        ],
        out_shape=[
            jax.ShapeDtypeStruct((N_PAD, D), jnp.float32),
            jax.ShapeDtypeStruct((8, N_PAD), jnp.float32),
        ],
    )(acc, den, b, W, a_s, a_d)


def _tc_final_body(acc_ref, den_ref, b_ref, o_ref):
    o_ref[...] = _combine(acc_ref, den_ref, b_ref)


def _tc_final(acc, den, b):
    return pl.pallas_call(
        _tc_final_body,
        grid=(_GRID,),
        in_specs=[
            pl.BlockSpec((NC, _BLK, D), lambda i: (0, i, 0)),
            pl.BlockSpec((NC, _BLK), lambda i: (0, i)),
            pl.BlockSpec((D,), lambda i: (0,)),
        ],
        out_specs=pl.BlockSpec((_BLK, D), lambda i: (i, 0)),
        out_shape=jax.ShapeDtypeStruct((N_PAD, D), jnp.float32),
    )(acc, den, b)


def _edge_pass_kernel(h_hbm, aa_hbm, src_hbm, dst_hbm,
                      acc_hbm, den_hbm,
                      acc_sh, den_sh, as_tab, ad_tab,
                      si_0, si_1, di_0, di_1, sdi_0, sdi_1,
                      rows_0, rows_1, w_0, w_1,
                      sem_g0, sem_g1, sem_s0, sem_s1, sem_d0, sem_d1,
                      sem_a0, sem_a1, sem_w0, sem_w1):
    cid = lax.axis_index("c")
    sid = lax.axis_index("s")
    wid = sid * NC + cid
    base = wid * T_PER_TILE

    zero16 = jnp.zeros((L,), jnp.float32)
    rows = (rows_0, rows_1)
    sis = (si_0, si_1)
    dis = (di_0, di_1)
    sdis = (sdi_0, sdi_1)
    w_v = (w_0, w_1)
    sems_g = (sem_g0, sem_g1)
    sems_s = (sem_s0, sem_s1)
    sems_d = (sem_d0, sem_d1)
    sems_a = (sem_a0, sem_a1)
    sems_w = (sem_w0, sem_w1)

    def _zrow(j, _):
        for c in range(D // L):
            rows_0[j, pl.ds(c * L, L)] = zero16
        return 0
    lax.fori_loop(0, K, _zrow, 0)
    for g in range(K // L):
        w_0[pl.ds(g * L, L)] = zero16

    ZB = 64
    zrow_src = rows_0.at[pl.ds(0, ZB)]
    zden_src = w_0.at[pl.ds(0, ZB)]
    for kk in range(ROWS_PER_TILE // ZB):
        off = sid * ROWS_PER_TILE + kk * ZB
        pltpu.async_copy(zrow_src, acc_sh.at[pl.ds(off, ZB)], sem_a0)
        pltpu.async_copy(zden_src, den_sh.at[pl.ds(off, ZB)], sem_w0)
    pltpu.async_copy(aa_hbm.at[0], as_tab, sem_g0)
    pltpu.async_copy(aa_hbm.at[1], ad_tab, sem_g1)
    for kk in range(ROWS_PER_TILE // ZB):
        off = sid * ROWS_PER_TILE + kk * ZB
        pltpu.make_async_copy(zrow_src, acc_sh.at[pl.ds(off, ZB)], sem_a0).wait()
        pltpu.make_async_copy(zden_src, den_sh.at[pl.ds(off, ZB)], sem_w0).wait()
    pltpu.make_async_copy(aa_hbm.at[0], as_tab, sem_g0).wait()
    pltpu.make_async_copy(aa_hbm.at[1], ad_tab, sem_g1).wait()

    plsc.subcore_barrier()

    def _issue_idx(c, b):
        sl = pl.ds(base + c * K, K)
        pltpu.async_copy(src_hbm.at[sl], sis[b], sems_s[b])
        pltpu.async_copy(dst_hbm.at[sl], dis[b], sems_d[b])

    def _wait_idx(c, b):
        sl = pl.ds(base + c * K, K)
        pltpu.make_async_copy(src_hbm.at[sl], sis[b], sems_s[b]).wait()
        pltpu.make_async_copy(dst_hbm.at[sl], dis[b], sems_d[b]).wait()

    def _issue_gather(c, b):
        pltpu.async_copy(h_hbm.at[sis[b]], rows[b], sems_g[b])

    def _wait_gather(c, b):
        pltpu.make_async_copy(h_hbm.at[sis[b]], rows[b], sems_g[b]).wait()

    def _process(c, b):
        wvb, sdb, sib, dib = w_v[b], sdis[b], sis[b], dis[b]

        @plsc.parallel_loop(0, K // L, 1, unroll=4)
        def _weights(g):
            s16 = sib[pl.ds(g * L, L)]
            d16 = dib[pl.ds(g * L, L)]
            e = plsc.load_gather(as_tab, [s16]) + plsc.load_gather(ad_tab, [d16])
            e = jnp.maximum(e, 0.2 * e)
            w = jnp.exp(e)
            wvb[pl.ds(g * L, L)] = w
            sdb[pl.ds(g * L, L)] = d16

        pltpu.async_copy(w_v[b], den_sh.at[sdis[b]], sems_w[b], add=True)

        rv = rows[b]
        wv = w_v[b]

        @plsc.parallel_loop(0, K, 1, unroll=4)
        def _scale(jj):
            j16 = jnp.full((L,), 0, jnp.int32) + jj
            wj = plsc.load_gather(wv, [j16])
            for c_ in range(D // L):
                rv[jj, pl.ds(c_ * L, L)] = rv[jj, pl.ds(c_ * L, L)] * wj

        pltpu.async_copy(rv, acc_sh.at[sdis[b]], sems_a[b], add=True)

    def _wait_scatters(c, b):
        pltpu.make_async_copy(w_v[b], den_sh.at[sdis[b]], sems_w[b]).wait()
        pltpu.make_async_copy(rows[b], acc_sh.at[sdis[b]], sems_a[b]).wait()

    _issue_idx(0, 0)
    _issue_idx(1, 1)
    _wait_idx(0, 0)
    _issue_gather(0, 0)
    _wait_idx(1, 1)
    _issue_gather(1, 1)
    _wait_gather(0, 0)
    _process(0, 0)
    _issue_idx(2, 0)

    def _step(c, b):
        nb = 1 - b
        _wait_scatters(c - 1, nb)
        _wait_idx(c + 1, nb)
        _issue_gather(c + 1, nb)
        _wait_gather(c, b)
        _process(c, b)
        _issue_idx(c + 2, b)

    def _pair(k, _):
        _step(2 * k + 1, 1)
        _step(2 * k + 2, 0)
        return 0

    lax.fori_loop(0, (N_CHUNKS - 2) // 2, _pair, 0)

    c_last = N_CHUNKS - 1
    _wait_scatters(c_last - 1, 0)
    _wait_gather(c_last, 1)
    _process(c_last, 1)
    _wait_scatters(c_last, 1)
    _wait_idx(N_CHUNKS, 0)

    plsc.subcore_barrier()

    stripe = pl.ds(sid * ROWS_PER_TILE, ROWS_PER_TILE)
    pltpu.sync_copy(acc_sh.at[stripe], acc_hbm.at[cid, stripe])
    pltpu.sync_copy(den_sh.at[stripe], den_hbm.at[cid, stripe])


def _edge_pass(h, aa, src, dst):
    mesh = plsc.VectorSubcoreMesh(core_axis_name="c", subcore_axis_name="s",
                                  num_cores=NC, num_subcores=NS)
    kern = pl.kernel(
        _edge_pass_kernel,
        out_type=(
            jax.ShapeDtypeStruct((NC, N_PAD, D), jnp.float32),
            jax.ShapeDtypeStruct((NC, N_PAD), jnp.float32),
        ),
        mesh=mesh,
        compiler_params=pltpu.CompilerParams(needs_layout_passes=False),
        scratch_types=(
            pltpu.VMEM_SHARED((N_PAD, D), jnp.float32),
            pltpu.VMEM_SHARED((N_PAD,), jnp.float32),
            pltpu.VMEM((N_PAD,), jnp.float32),
            pltpu.VMEM((N_PAD,), jnp.float32),
            pltpu.VMEM((K,), jnp.int32),
            pltpu.VMEM((K,), jnp.int32),
            pltpu.VMEM((K,), jnp.int32),
            pltpu.VMEM((K,), jnp.int32),
            pltpu.VMEM((K,), jnp.int32),
            pltpu.VMEM((K,), jnp.int32),
            pltpu.VMEM((K, D), jnp.float32),
            pltpu.VMEM((K, D), jnp.float32),
            pltpu.VMEM((K,), jnp.float32),
            pltpu.VMEM((K,), jnp.float32),
            pltpu.SemaphoreType.DMA,
            pltpu.SemaphoreType.DMA,
            pltpu.SemaphoreType.DMA,
            pltpu.SemaphoreType.DMA,
            pltpu.SemaphoreType.DMA,
            pltpu.SemaphoreType.DMA,
            pltpu.SemaphoreType.DMA,
            pltpu.SemaphoreType.DMA,
            pltpu.SemaphoreType.DMA,
            pltpu.SemaphoreType.DMA,
        ),
    )
    return kern(h, aa, src, dst)


def kernel(x_note, edge_index, edge_attr, W1, a_src1, a_dst1, b1,
           W2, a_src2, a_dst2, b2):
    del edge_attr
    x_pad = jnp.zeros((N_PAD, D), jnp.float32).at[:N].set(x_note)

    loops = jnp.arange(N, dtype=jnp.int32)
    pad = jnp.full((E_HBM - E_TOT,), N_PAD - 1, jnp.int32)
    src = jnp.concatenate([edge_index[0].astype(jnp.int32), loops, pad])
    dst = jnp.concatenate([edge_index[1].astype(jnp.int32), loops, pad])

    h1, aa1 = _tc_prep(x_pad, W1, a_src1, a_dst1)
    acc1, den1 = _edge_pass(h1, aa1, src, dst)
    h2, aa2 = _tc_mid(acc1, den1, b1, W2, a_src2, a_dst2)
    acc2, den2 = _edge_pass(h2, aa2, src, dst)
    out = _tc_final(acc2, den2, b2)
    return out[:N]
